# Initial kernel scaffold; baseline (speedup 1.0000x reference)
import jax
import jax.numpy as jnp
from jax.experimental import pallas as pl


def kernel(x, edge_index, batch, edge_weights, W1, b1, W2, b2, W3, b3, Wlin, blin):
    raise NotImplementedError("write your pallas kernel here")


# speedup vs baseline: 4.5429x; 4.5429x over previous
import dataclasses
import functools

import jax
import jax.numpy as jnp
from jax import lax
from jax.experimental import pallas as pl
from jax.experimental.pallas import tpu as pltpu
from jax.experimental.pallas import tpu_sc as plsc

N_NODES = 10000
N_EDGES = 320000
N_GRAPHS = 64
N_CLASSES = 10

_NC = 2
_NS = 16
_NW = _NC * _NS
_K = 128
_N_CHUNKS = N_EDGES // _K
_CHUNKS_PER_W = (_N_CHUNKS + _NW - 1) // _NW
_ROWS_PER_TILE = 624
_ROWS_REM = N_NODES - _NS * _ROWS_PER_TILE

_HIGH = lax.Precision.HIGHEST


def _dot(a, b):
    return lax.dot_general(a, b, (((1,), (0,)), ((), ())),
                           preferred_element_type=jnp.float32,
                           precision=_HIGH)


def _mm(x, w):
    def body(x_ref, w_ref, o_ref):
        o_ref[...] = _dot(x_ref[...], w_ref[...])
    return pl.pallas_call(
        body,
        out_shape=jax.ShapeDtypeStruct((x.shape[0], w.shape[1]), jnp.float32),
    )(x, w)


def _fuse(acc, b, w):
    def body(a_ref, b_ref, w_ref, o_ref):
        h = jnp.maximum(a_ref[0] + a_ref[1] + b_ref[...], 0.0)
        o_ref[...] = _dot(h, w_ref[...])
    return pl.pallas_call(
        body,
        out_shape=jax.ShapeDtypeStruct((acc.shape[1], w.shape[1]), jnp.float32),
    )(acc, b.reshape(1, -1), w)


def _final(acc, b, batch2d, wlin, blin):
    def body(a_ref, b_ref, bt_ref, wl_ref, bl_ref, o_ref):
        out3 = a_ref[0] + a_ref[1] + b_ref[...]
        gi = lax.broadcasted_iota(jnp.int32, (N_NODES, N_GRAPHS), 1)
        onehot = (bt_ref[...] == gi).astype(jnp.float32)
        sums = lax.dot_general(onehot, out3, (((0,), (0,)), ((), ())),
                               preferred_element_type=jnp.float32,
                               precision=_HIGH)
        ones = jnp.ones((N_NODES, 1), jnp.float32)
        counts = lax.dot_general(onehot, ones, (((0,), (0,)), ((), ())),
                                 preferred_element_type=jnp.float32,
                                 precision=_HIGH)
        pooled = sums / jnp.maximum(counts, 1.0)
        o_ref[...] = _dot(pooled, wl_ref[...]) + bl_ref[...]
    return pl.pallas_call(
        body,
        out_shape=jax.ShapeDtypeStruct((N_GRAPHS, N_CLASSES), jnp.float32),
    )(acc, b.reshape(1, -1), batch2d, wlin, blin.reshape(1, -1))


def _make_scatter(d):
    mesh = plsc.VectorSubcoreMesh(core_axis_name="c", subcore_axis_name="s")
    cp = pltpu.CompilerParams()
    if "needs_layout_passes" in pltpu.CompilerParams.__dataclass_fields__:
        cp = dataclasses.replace(cp, needs_layout_passes=False)
    if d < 128 and "use_tc_tiling_on_sc" in pltpu.CompilerParams.__dataclass_fields__:
        cp = dataclasses.replace(cp, use_tc_tiling_on_sc=False)

    @functools.partial(
        pl.kernel,
        compiler_params=cp,
        out_type=jax.ShapeDtypeStruct((_NC, N_NODES, d), jnp.float32),
        mesh=mesh,
        scratch_types=[
            pltpu.VMEM((_K,), jnp.int32),
            pltpu.VMEM((_K,), jnp.int32),
            pltpu.VMEM((_K,), jnp.float32),
            pltpu.VMEM((_K, d), jnp.float32),
            pltpu.VMEM_SHARED((N_NODES, d), jnp.float32),
            pltpu.SemaphoreType.DMA,
        ],
    )
    def sc_kernel(h_hbm, src_hbm, dst_hbm, w_hbm, z_hbm, out_hbm,
                  srcv, dstv, wv, rows, acc, sem):
        c = lax.axis_index("c")
        s = lax.axis_index("s")
        wid = s * _NC + c
        r0 = s * _ROWS_PER_TILE

        pltpu.sync_copy(z_hbm.at[pl.ds(r0, _ROWS_PER_TILE)],
                        acc.at[pl.ds(r0, _ROWS_PER_TILE)])

        @pl.when(s == _NS - 1)
        def _():
            pltpu.sync_copy(z_hbm.at[pl.ds(_NS * _ROWS_PER_TILE, _ROWS_REM)],
                            acc.at[pl.ds(_NS * _ROWS_PER_TILE, _ROWS_REM)])

        plsc.subcore_barrier()

        @pl.loop(0, _CHUNKS_PER_W)
        def _(i):
            ci = i * _NW + wid

            @pl.when(ci < _N_CHUNKS)
            def _():
                e0 = ci * _K
                pltpu.sync_copy(src_hbm.at[pl.ds(e0, _K)], srcv)
                pltpu.sync_copy(dst_hbm.at[pl.ds(e0, _K)], dstv)
                pltpu.sync_copy(w_hbm.at[pl.ds(e0, _K)], wv)
                pltpu.async_copy(h_hbm.at[srcv], rows, sem).wait()

                @pl.loop(0, _K)
                def _(k):
                    wb = plsc.load_gather(wv, [jnp.full((16,), k, jnp.int32)])
                    for j in range(d // 16):
                        sl = (k, pl.ds(j * 16, 16))
                        rows[sl] = rows[sl] * wb

                pltpu.sync_copy(rows, acc.at[dstv], add=True)

        plsc.subcore_barrier()
        pltpu.sync_copy(acc.at[pl.ds(r0, _ROWS_PER_TILE)],
                        out_hbm.at[c, pl.ds(r0, _ROWS_PER_TILE)])

        @pl.when(s == _NS - 1)
        def _():
            pltpu.sync_copy(acc.at[pl.ds(_NS * _ROWS_PER_TILE, _ROWS_REM)],
                            out_hbm.at[c, pl.ds(_NS * _ROWS_PER_TILE, _ROWS_REM)])

    return sc_kernel


_scatter128 = _make_scatter(128)
_scatter64 = _make_scatter(64)


@jax.jit
def kernel(x, edge_index, batch, edge_weights, W1, b1, W2, b2, W3, b3,
           Wlin, blin):
    src = edge_index[0].astype(jnp.int32)
    dst = edge_index[1].astype(jnp.int32)
    z128 = jnp.zeros((N_NODES, 128), jnp.float32)
    z64 = jnp.zeros((N_NODES, 64), jnp.float32)
    batch2d = batch.astype(jnp.int32).reshape(N_NODES, 1)

    h1 = _mm(x, W1)
    a1 = _scatter128(h1, src, dst, edge_weights, z128)
    h2 = _fuse(a1, b1, W2)
    a2 = _scatter128(h2, src, dst, edge_weights, z128)
    h3 = _fuse(a2, b2, W3)
    a3 = _scatter64(h3, src, dst, edge_weights, z64)
    return _final(a3, b3, batch2d, Wlin, blin)
